# Initial kernel scaffold; baseline (speedup 1.0000x reference)
import jax
import jax.numpy as jnp
from jax.experimental import pallas as pl


def kernel(InState, NNsites, GnnPerms, SitesToShells, gdiags, Psi0, b0, Psi1, b1, Psi2, b2, Psi3, b3, Psi4, b4, PsiR, VR):
    raise NotImplementedError("write your pallas kernel here")


# speedup vs baseline: 15.0284x; 15.0284x over previous
import functools

import jax
import jax.numpy as jnp
from jax import lax
from jax.experimental import pallas as pl
from jax.experimental.pallas import tpu as pltpu
from jax.experimental.pallas import tpu_sc as plsc

_B = 4
_NSPEC = 5
_S = 8192
_NN = 13
_NG = 48
_NSH = 6
_DIM = 3
_C = 8
_NC = 2
_NS = 16
_NW = _NC * _NS
_BLK = 512


def _sc_gather(table, idx2d, d):
    n128 = idx2d.shape[0]
    per_w = n128 // _NW
    u = 13
    nsup = per_w // u
    assert per_w == nsup * u

    mesh = plsc.VectorSubcoreMesh(core_axis_name="c", subcore_axis_name="s")

    @functools.partial(
        pl.kernel,
        mesh=mesh,
        compiler_params=pltpu.CompilerParams(use_tc_tiling_on_sc=False),
        out_type=jax.ShapeDtypeStruct((n128, 128, d), jnp.float32),
        scratch_types=[
            pltpu.VMEM((per_w, 128), jnp.int32),
            pltpu.VMEM((per_w, 128, d), jnp.float32),
            pltpu.SemaphoreType.DMA,
        ],
    )
    def k(table_hbm, idx_hbm, out_hbm, idx_v, rows_v, sem):
        wid = lax.axis_index("s") * _NC + lax.axis_index("c")
        base = wid * per_w
        pltpu.sync_copy(idx_hbm.at[pl.ds(base, per_w)], idx_v)

        def sup(j, carry):
            cps = []
            for t in range(u):
                r = j * u + t
                cps.append(pltpu.async_copy(
                    table_hbm.at[idx_v.at[r]], rows_v.at[r], sem))
            for cp in cps:
                cp.wait()
            return carry

        lax.fori_loop(0, nsup, sup, 0)
        pltpu.sync_copy(rows_v, out_hbm.at[pl.ds(base, per_w)])

    return k(table, idx2d)


def _softplus(x):
    return jnp.maximum(x, 0.0) + jnp.log(1.0 + jnp.exp(-jnp.abs(x)))


def _conv_layer(g, p2, bvec, mm):
    ncol = p2.shape[1]
    o = mm.shape[1]
    out_shape = jax.ShapeDtypeStruct((_B, _S, o), jnp.float32)
    out_spec = pl.BlockSpec((1, _BLK, o), lambda b, j: (b, j, 0))

    def body(g_ref, p_ref, b_ref, m_ref, o_ref):
        x = jnp.dot(g_ref[0], p_ref[...], preferred_element_type=jnp.float32,
                        precision=lax.Precision.HIGHEST)
        x = x + b_ref[...]
        o_ref[0] = jnp.dot(_softplus(x), m_ref[...],
                           preferred_element_type=jnp.float32,
                        precision=lax.Precision.HIGHEST)

    return pl.pallas_call(
        body,
        grid=(_B, _S // _BLK),
        in_specs=[
            pl.BlockSpec((1, _BLK, _NN * _C), lambda b, j: (b, j, 0)),
            pl.BlockSpec((_NN * _C, ncol), lambda b, j: (0, 0)),
            pl.BlockSpec((1, ncol), lambda b, j: (0, 0)),
            pl.BlockSpec((ncol, o), lambda b, j: (0, 0)),
        ],
        out_specs=out_spec,
        out_shape=out_shape,
    )(g, p2, bvec, mm)


def _conv_last(g, p2, bvec, mm):
    ncol = p2.shape[1]

    def body(g_ref, p_ref, b_ref, m_ref, o_ref):
        cols = []
        for b in range(_B):
            x = jnp.dot(g_ref[b], p_ref[...],
                        preferred_element_type=jnp.float32,
                        precision=lax.Precision.HIGHEST)
            x = x + b_ref[...]
            cols.append(jnp.dot(_softplus(x), m_ref[...],
                                preferred_element_type=jnp.float32,
                        precision=lax.Precision.HIGHEST))
        cols.append(jnp.zeros((_BLK, _C - _B), jnp.float32))
        o_ref[...] = jnp.concatenate(cols, axis=1)

    return pl.pallas_call(
        body,
        grid=(_S // _BLK,),
        in_specs=[
            pl.BlockSpec((_B, _BLK, _NN * _C), lambda j: (0, j, 0)),
            pl.BlockSpec((_NN * _C, ncol), lambda j: (0, 0)),
            pl.BlockSpec((1, ncol), lambda j: (0, 0)),
            pl.BlockSpec((ncol, 1), lambda j: (0, 0)),
        ],
        out_specs=pl.BlockSpec((_BLK, _C), lambda j: (j, 0)),
        out_shape=jax.ShapeDtypeStruct((_S, _C), jnp.float32),
    )(g, p2, bvec, mm)


def _readout_body(g_ref, sh_ref, k_ref, o_ref):
    blk = g_ref.shape[0]
    oh_t = (lax.broadcasted_iota(jnp.int32, (_NSH, blk), 0) ==
            sh_ref[0]).astype(jnp.float32)
    w2 = lax.dot_general(oh_t, k_ref[...], (((0,), (0,)), ((), ())),
                         preferred_element_type=jnp.float32,
                        precision=lax.Precision.HIGHEST)
    d = lax.dot_general(g_ref[...], w2, (((0,), (0,)), ((), ())),
                        preferred_element_type=jnp.float32,
                        precision=lax.Precision.HIGHEST)

    @pl.when(pl.program_id(0) == 0)
    def _init():
        o_ref[...] = jnp.zeros_like(o_ref)

    o_ref[...] += d


def _readout(g5, shells3, k6):
    return pl.pallas_call(
        _readout_body,
        grid=(_S // _BLK,),
        in_specs=[
            pl.BlockSpec((_BLK, _NN * _C), lambda j: (j, 0)),
            pl.BlockSpec((1, 1, _BLK), lambda j: (j, 0, 0)),
            pl.BlockSpec((_NSH, _NN * _DIM), lambda j: (0, 0)),
        ],
        out_specs=pl.BlockSpec((_NN * _C, _NN * _DIM), lambda j: (0, 0)),
        out_shape=jax.ShapeDtypeStruct((_NN * _C, _NN * _DIM), jnp.float32),
    )(g5, shells3, k6)


def _p2_of(psi, gnnperms):
    psig = psi[:, :, gnnperms]
    p = jnp.transpose(psig, (3, 1, 0, 2))
    cin = p.shape[1]
    if cin < _C:
        p = jnp.pad(p, ((0, 0), (0, _C - cin), (0, 0), (0, 0)))
    o = p.shape[2]
    return p.reshape(_NN * _C, o * _NG)


def kernel(InState, NNsites, GnnPerms, SitesToShells, gdiags,
           Psi0, b0, Psi1, b1, Psi2, b2, Psi3, b3, Psi4, b4, PsiR, VR):
    psis = (Psi0, Psi1, Psi2, Psi3, Psi4)
    biases = (b0, b1, b2, b3, b4)

    nnt = NNsites.T.astype(jnp.int32)
    idx5 = nnt.reshape(_S * _NN // 128, 128)
    idxc = (jnp.arange(_B, dtype=jnp.int32)[:, None, None] * _S
            + nnt[None]).reshape(_B * _S * _NN // 128, 128)

    p2s = [_p2_of(p, GnnPerms) for p in psis]
    bvecs = [jnp.repeat(b, _NG)[None, :] for b in biases]
    mms = [jnp.kron(jnp.eye(p.shape[0], dtype=jnp.float32),
                    jnp.full((_NG, 1), 1.0 / _NG, jnp.float32))
           for p in psis]

    q = jnp.einsum('hd,gde->hge', VR, gdiags)
    psirg = PsiR[:, GnnPerms]
    k6 = (jnp.einsum('hgn,hge->hne', psirg, q) / _NG).reshape(_NSH, _NN * _DIM)

    shells3 = SitesToShells.astype(jnp.int32).reshape(_S // _BLK, 1, _BLK)

    x = jnp.transpose(InState, (0, 2, 1))
    x = jnp.pad(x, ((0, 0), (0, 0), (0, _C - _NSPEC)))

    for li in range(5):
        table = x.reshape(_B * _S, _C)
        g = _sc_gather(table, idxc, _C)
        g = g.reshape(_B, _S, _NN * _C)
        if li < 4:
            x = _conv_layer(g, p2s[li], bvecs[li], mms[li])
        else:
            x = _conv_last(g, p2s[li], bvecs[li], mms[li])

    g5 = _sc_gather(x, idx5, _C).reshape(_S, _NN * _C)
    dmat = _readout(g5, shells3, k6)

    d4 = dmat.reshape(_NN, _C, _NN, _DIM)
    return jnp.einsum('nbne->be', d4)[:_B]


# speedup vs baseline: 15.8482x; 1.0545x over previous
import functools

import jax
import jax.numpy as jnp
from jax import lax
from jax.experimental import pallas as pl
from jax.experimental.pallas import tpu as pltpu
from jax.experimental.pallas import tpu_sc as plsc

_B = 4
_NSPEC = 5
_S = 8192
_NN = 13
_NG = 48
_NSH = 6
_DIM = 3
_C = 8
_NC = 2
_NS = 16
_NW = _NC * _NS
_BLK = 512


def _sc_gather(table, idx2d, d):
    n128 = idx2d.shape[0]
    per_w = n128 // _NW
    u = 13
    nsup = per_w // u
    assert per_w == nsup * u

    mesh = plsc.VectorSubcoreMesh(core_axis_name="c", subcore_axis_name="s")

    @functools.partial(
        pl.kernel,
        mesh=mesh,
        compiler_params=pltpu.CompilerParams(use_tc_tiling_on_sc=False),
        out_type=jax.ShapeDtypeStruct((n128, 128, d), jnp.float32),
        scratch_types=[
            pltpu.VMEM((per_w, 128), jnp.int32),
            pltpu.VMEM((per_w, 128, d), jnp.float32),
            pltpu.SemaphoreType.DMA,
        ],
    )
    def k(table_hbm, idx_hbm, out_hbm, idx_v, rows_v, sem):
        wid = lax.axis_index("s") * _NC + lax.axis_index("c")
        base = wid * per_w
        pltpu.sync_copy(idx_hbm.at[pl.ds(base, per_w)], idx_v)

        def sup(j, carry):
            cps = []
            for t in range(u):
                r = j * u + t
                cps.append(pltpu.async_copy(
                    table_hbm.at[idx_v.at[r]], rows_v.at[r], sem))
            for cp in cps:
                cp.wait()
            return carry

        lax.fori_loop(0, nsup, sup, 0)
        pltpu.sync_copy(rows_v, out_hbm.at[pl.ds(base, per_w)])

    return k(table, idx2d)


def _softplus(x):
    return jnp.maximum(x, 0.0) + jnp.log(1.0 + jnp.exp(-jnp.abs(x)))


def _conv_layer(g, p2, bvec, mm):
    ncol = p2.shape[1]
    o = mm.shape[1]

    def body(g_ref, p_ref, b_ref, m_ref, o_ref):
        gm = g_ref[...].reshape(_B * _BLK, _NN * _C)
        x = jnp.dot(gm, p_ref[...], preferred_element_type=jnp.float32,
                    precision=lax.Precision.HIGHEST)
        x = x + b_ref[...]
        y = jnp.dot(_softplus(x), m_ref[...],
                    preferred_element_type=jnp.float32,
                    precision=lax.Precision.HIGHEST)
        o_ref[...] = y.reshape(_B, _BLK, o)

    return pl.pallas_call(
        body,
        grid=(_S // _BLK,),
        in_specs=[
            pl.BlockSpec((_B, _BLK, _NN * _C), lambda j: (0, j, 0)),
            pl.BlockSpec((_NN * _C, ncol), lambda j: (0, 0)),
            pl.BlockSpec((1, ncol), lambda j: (0, 0)),
            pl.BlockSpec((ncol, o), lambda j: (0, 0)),
        ],
        out_specs=pl.BlockSpec((_B, _BLK, o), lambda j: (0, j, 0)),
        out_shape=jax.ShapeDtypeStruct((_B, _S, o), jnp.float32),
    )(g, p2, bvec, mm)


def _conv_last(g, p2, bvec, mm):
    ncol = p2.shape[1]

    def body(g_ref, p_ref, b_ref, m_ref, o_ref):
        gm = g_ref[...].reshape(_B * _BLK, _NN * _C)
        x = jnp.dot(gm, p_ref[...], preferred_element_type=jnp.float32,
                    precision=lax.Precision.HIGHEST)
        x = x + b_ref[...]
        y = jnp.dot(_softplus(x), m_ref[...],
                    preferred_element_type=jnp.float32,
                    precision=lax.Precision.HIGHEST)
        y = y.reshape(_B, _BLK, 1)
        cols = [y[b] for b in range(_B)]
        cols.append(jnp.zeros((_BLK, _C - _B), jnp.float32))
        o_ref[...] = jnp.concatenate(cols, axis=1)

    return pl.pallas_call(
        body,
        grid=(_S // _BLK,),
        in_specs=[
            pl.BlockSpec((_B, _BLK, _NN * _C), lambda j: (0, j, 0)),
            pl.BlockSpec((_NN * _C, ncol), lambda j: (0, 0)),
            pl.BlockSpec((1, ncol), lambda j: (0, 0)),
            pl.BlockSpec((ncol, 1), lambda j: (0, 0)),
        ],
        out_specs=pl.BlockSpec((_BLK, _C), lambda j: (j, 0)),
        out_shape=jax.ShapeDtypeStruct((_S, _C), jnp.float32),
    )(g, p2, bvec, mm)


def _readout_body(g_ref, sh_ref, k_ref, o_ref):
    blk = g_ref.shape[0]
    oh_t = (lax.broadcasted_iota(jnp.int32, (_NSH, blk), 0) ==
            sh_ref[0]).astype(jnp.float32)
    w2 = lax.dot_general(oh_t, k_ref[...], (((0,), (0,)), ((), ())),
                         preferred_element_type=jnp.float32,
                        precision=lax.Precision.HIGHEST)
    d = lax.dot_general(g_ref[...], w2, (((0,), (0,)), ((), ())),
                        preferred_element_type=jnp.float32,
                        precision=lax.Precision.HIGHEST)

    @pl.when(pl.program_id(0) == 0)
    def _init():
        o_ref[...] = jnp.zeros_like(o_ref)

    o_ref[...] += d


def _readout(g5, shells3, k6):
    return pl.pallas_call(
        _readout_body,
        grid=(_S // _BLK,),
        in_specs=[
            pl.BlockSpec((_BLK, _NN * _C), lambda j: (j, 0)),
            pl.BlockSpec((1, 1, _BLK), lambda j: (j, 0, 0)),
            pl.BlockSpec((_NSH, _NN * _DIM), lambda j: (0, 0)),
        ],
        out_specs=pl.BlockSpec((_NN * _C, _NN * _DIM), lambda j: (0, 0)),
        out_shape=jax.ShapeDtypeStruct((_NN * _C, _NN * _DIM), jnp.float32),
    )(g5, shells3, k6)


def _p2_of(psi, gnnperms):
    psig = psi[:, :, gnnperms]
    p = jnp.transpose(psig, (3, 1, 0, 2))
    cin = p.shape[1]
    if cin < _C:
        p = jnp.pad(p, ((0, 0), (0, _C - cin), (0, 0), (0, 0)))
    o = p.shape[2]
    return p.reshape(_NN * _C, o * _NG)


def kernel(InState, NNsites, GnnPerms, SitesToShells, gdiags,
           Psi0, b0, Psi1, b1, Psi2, b2, Psi3, b3, Psi4, b4, PsiR, VR):
    psis = (Psi0, Psi1, Psi2, Psi3, Psi4)
    biases = (b0, b1, b2, b3, b4)

    nnt = NNsites.T.astype(jnp.int32)
    idx5 = nnt.reshape(_S * _NN // 128, 128)
    idxc = (jnp.arange(_B, dtype=jnp.int32)[:, None, None] * _S
            + nnt[None]).reshape(_B * _S * _NN // 128, 128)

    p2s = [_p2_of(p, GnnPerms) for p in psis]
    bvecs = [jnp.repeat(b, _NG)[None, :] for b in biases]
    mms = [jnp.kron(jnp.eye(p.shape[0], dtype=jnp.float32),
                    jnp.full((_NG, 1), 1.0 / _NG, jnp.float32))
           for p in psis]

    q = jnp.einsum('hd,gde->hge', VR, gdiags)
    psirg = PsiR[:, GnnPerms]
    k6 = (jnp.einsum('hgn,hge->hne', psirg, q) / _NG).reshape(_NSH, _NN * _DIM)

    shells3 = SitesToShells.astype(jnp.int32).reshape(_S // _BLK, 1, _BLK)

    x = jnp.transpose(InState, (0, 2, 1))
    x = jnp.pad(x, ((0, 0), (0, 0), (0, _C - _NSPEC)))

    for li in range(5):
        table = x.reshape(_B * _S, _C)
        g = _sc_gather(table, idxc, _C)
        g = g.reshape(_B, _S, _NN * _C)
        if li < 4:
            x = _conv_layer(g, p2s[li], bvecs[li], mms[li])
        else:
            x = _conv_last(g, p2s[li], bvecs[li], mms[li])

    g5 = _sc_gather(x, idx5, _C).reshape(_S, _NN * _C)
    dmat = _readout(g5, shells3, k6)

    d4 = dmat.reshape(_NN, _C, _NN, _DIM)
    return jnp.einsum('nbne->be', d4)[:_B]


# speedup vs baseline: 22.7784x; 1.4373x over previous
import functools

import jax
import jax.numpy as jnp
from jax import lax
from jax.experimental import pallas as pl
from jax.experimental.pallas import tpu as pltpu
from jax.experimental.pallas import tpu_sc as plsc

_B = 4
_NSPEC = 5
_S = 8192
_NN = 13
_NG = 48
_NSH = 6
_DIM = 3
_C = 8
_NC = 2
_NS = 16
_NW = _NC * _NS
_BLK = 512


def _sc_gather(table, idx2d, d):
    n128 = idx2d.shape[0]
    per_w = n128 // _NW
    u = 13
    nsup = per_w // u
    assert per_w == nsup * u

    mesh = plsc.VectorSubcoreMesh(core_axis_name="c", subcore_axis_name="s")

    @functools.partial(
        pl.kernel,
        mesh=mesh,
        compiler_params=pltpu.CompilerParams(use_tc_tiling_on_sc=False),
        out_type=jax.ShapeDtypeStruct((n128, 128, d), jnp.float32),
        scratch_types=[
            pltpu.VMEM((per_w, 128), jnp.int32),
            pltpu.VMEM((per_w, 128, d), jnp.float32),
            pltpu.SemaphoreType.DMA,
        ],
    )
    def k(table_hbm, idx_hbm, out_hbm, idx_v, rows_v, sem):
        wid = lax.axis_index("s") * _NC + lax.axis_index("c")
        base = wid * per_w
        pltpu.sync_copy(idx_hbm.at[pl.ds(base, per_w)], idx_v)

        def sup(j, carry):
            cps = []
            for t in range(u):
                r = j * u + t
                cps.append(pltpu.async_copy(
                    table_hbm.at[idx_v.at[r]], rows_v.at[r], sem))
            for cp in cps:
                cp.wait()
            return carry

        lax.fori_loop(0, nsup, sup, 0)
        pltpu.sync_copy(rows_v, out_hbm.at[pl.ds(base, per_w)])

    return k(table, idx2d)


def _softplus(x):
    return jnp.maximum(x, 0.0) + jnp.log(1.0 + jnp.exp(-jnp.abs(x)))


def _split_bf16(x):
    hi = x.astype(jnp.bfloat16)
    lo = (x - hi.astype(jnp.float32)).astype(jnp.bfloat16)
    return hi, lo


def _dot3(a, b):
    a_hi, a_lo = _split_bf16(a)
    b_hi, b_lo = _split_bf16(b)
    return (jnp.dot(a_hi, b_hi, preferred_element_type=jnp.float32)
            + (jnp.dot(a_hi, b_lo, preferred_element_type=jnp.float32)
               + jnp.dot(a_lo, b_hi, preferred_element_type=jnp.float32)))


def _group_mean(sp, m_ones):
    s_hi, s_lo = _split_bf16(sp)
    mb = m_ones.astype(jnp.bfloat16)
    return (jnp.dot(s_hi, mb, preferred_element_type=jnp.float32)
            + jnp.dot(s_lo, mb, preferred_element_type=jnp.float32)
            ) * (1.0 / _NG)


def _conv_layer(g, p2, bvec, mm):
    ncol = p2.shape[1]
    o = mm.shape[1]

    def body(g_ref, p_ref, b_ref, m_ref, o_ref):
        gm = g_ref[...].reshape(_B * _BLK, _NN * _C)
        x = _dot3(gm, p_ref[...]) + b_ref[...]
        y = _group_mean(_softplus(x), m_ref[...])
        o_ref[...] = y.reshape(_B, _BLK, o)

    return pl.pallas_call(
        body,
        grid=(_S // _BLK,),
        in_specs=[
            pl.BlockSpec((_B, _BLK, _NN * _C), lambda j: (0, j, 0)),
            pl.BlockSpec((_NN * _C, ncol), lambda j: (0, 0)),
            pl.BlockSpec((1, ncol), lambda j: (0, 0)),
            pl.BlockSpec((ncol, o), lambda j: (0, 0)),
        ],
        out_specs=pl.BlockSpec((_B, _BLK, o), lambda j: (0, j, 0)),
        out_shape=jax.ShapeDtypeStruct((_B, _S, o), jnp.float32),
    )(g, p2, bvec, mm)


def _conv_last(g, p2, bvec, mm):
    ncol = p2.shape[1]

    def body(g_ref, p_ref, b_ref, m_ref, o_ref):
        gm = g_ref[...].reshape(_B * _BLK, _NN * _C)
        x = _dot3(gm, p_ref[...]) + b_ref[...]
        y = _group_mean(_softplus(x), m_ref[...])
        y = y.reshape(_B, _BLK, 1)
        cols = [y[b] for b in range(_B)]
        cols.append(jnp.zeros((_BLK, _C - _B), jnp.float32))
        o_ref[...] = jnp.concatenate(cols, axis=1)

    return pl.pallas_call(
        body,
        grid=(_S // _BLK,),
        in_specs=[
            pl.BlockSpec((_B, _BLK, _NN * _C), lambda j: (0, j, 0)),
            pl.BlockSpec((_NN * _C, ncol), lambda j: (0, 0)),
            pl.BlockSpec((1, ncol), lambda j: (0, 0)),
            pl.BlockSpec((ncol, 1), lambda j: (0, 0)),
        ],
        out_specs=pl.BlockSpec((_BLK, _C), lambda j: (j, 0)),
        out_shape=jax.ShapeDtypeStruct((_S, _C), jnp.float32),
    )(g, p2, bvec, mm)


def _readout_body(g_ref, sh_ref, k_ref, o_ref):
    blk = g_ref.shape[0]
    oh_t = (lax.broadcasted_iota(jnp.int32, (_NSH, blk), 0) ==
            sh_ref[0]).astype(jnp.float32)
    w2 = lax.dot_general(oh_t, k_ref[...], (((0,), (0,)), ((), ())),
                         preferred_element_type=jnp.float32,
                        precision=lax.Precision.HIGHEST)
    d = lax.dot_general(g_ref[...], w2, (((0,), (0,)), ((), ())),
                        preferred_element_type=jnp.float32,
                        precision=lax.Precision.HIGHEST)

    @pl.when(pl.program_id(0) == 0)
    def _init():
        o_ref[...] = jnp.zeros_like(o_ref)

    o_ref[...] += d


def _readout(g5, shells3, k6):
    return pl.pallas_call(
        _readout_body,
        grid=(_S // _BLK,),
        in_specs=[
            pl.BlockSpec((_BLK, _NN * _C), lambda j: (j, 0)),
            pl.BlockSpec((1, 1, _BLK), lambda j: (j, 0, 0)),
            pl.BlockSpec((_NSH, _NN * _DIM), lambda j: (0, 0)),
        ],
        out_specs=pl.BlockSpec((_NN * _C, _NN * _DIM), lambda j: (0, 0)),
        out_shape=jax.ShapeDtypeStruct((_NN * _C, _NN * _DIM), jnp.float32),
    )(g5, shells3, k6)


def _p2_of(psi, gnnperms):
    psig = psi[:, :, gnnperms]
    p = jnp.transpose(psig, (3, 1, 0, 2))
    cin = p.shape[1]
    if cin < _C:
        p = jnp.pad(p, ((0, 0), (0, _C - cin), (0, 0), (0, 0)))
    o = p.shape[2]
    return p.reshape(_NN * _C, o * _NG)


def kernel(InState, NNsites, GnnPerms, SitesToShells, gdiags,
           Psi0, b0, Psi1, b1, Psi2, b2, Psi3, b3, Psi4, b4, PsiR, VR):
    psis = (Psi0, Psi1, Psi2, Psi3, Psi4)
    biases = (b0, b1, b2, b3, b4)

    nnt = NNsites.T.astype(jnp.int32)
    idx5 = nnt.reshape(_S * _NN // 128, 128)
    idxc = (jnp.arange(_B, dtype=jnp.int32)[:, None, None] * _S
            + nnt[None]).reshape(_B * _S * _NN // 128, 128)

    p2s = [_p2_of(p, GnnPerms) for p in psis]
    bvecs = [jnp.repeat(b, _NG)[None, :] for b in biases]
    mms = [jnp.kron(jnp.eye(p.shape[0], dtype=jnp.float32),
                    jnp.ones((_NG, 1), jnp.float32))
           for p in psis]

    q = jnp.einsum('hd,gde->hge', VR, gdiags)
    psirg = PsiR[:, GnnPerms]
    k6 = (jnp.einsum('hgn,hge->hne', psirg, q) / _NG).reshape(_NSH, _NN * _DIM)

    shells3 = SitesToShells.astype(jnp.int32).reshape(_S // _BLK, 1, _BLK)

    x = jnp.transpose(InState, (0, 2, 1))
    x = jnp.pad(x, ((0, 0), (0, 0), (0, _C - _NSPEC)))

    for li in range(5):
        table = x.reshape(_B * _S, _C)
        g = _sc_gather(table, idxc, _C)
        g = g.reshape(_B, _S, _NN * _C)
        if li < 4:
            x = _conv_layer(g, p2s[li], bvecs[li], mms[li])
        else:
            x = _conv_last(g, p2s[li], bvecs[li], mms[li])

    g5 = _sc_gather(x, idx5, _C).reshape(_S, _NN * _C)
    dmat = _readout(g5, shells3, k6)

    d4 = dmat.reshape(_NN, _C, _NN, _DIM)
    return jnp.einsum('nbne->be', d4)[:_B]


# speedup vs baseline: 25.5732x; 1.1227x over previous
import functools

import jax
import jax.numpy as jnp
from jax import lax
from jax.experimental import pallas as pl
from jax.experimental.pallas import tpu as pltpu
from jax.experimental.pallas import tpu_sc as plsc

_B = 4
_NSPEC = 5
_S = 8192
_NN = 13
_NG = 48
_NSH = 6
_DIM = 3
_C = 8
_NC = 2
_NS = 16
_NW = _NC * _NS
_BLK = 512


def _sc_gather(table, idx2d, d):
    n128 = idx2d.shape[0]
    per_w = n128 // _NW
    u = 13
    nsup = per_w // u
    assert per_w == nsup * u

    mesh = plsc.VectorSubcoreMesh(core_axis_name="c", subcore_axis_name="s")

    @functools.partial(
        pl.kernel,
        mesh=mesh,
        compiler_params=pltpu.CompilerParams(use_tc_tiling_on_sc=False),
        out_type=jax.ShapeDtypeStruct((n128, 128, d), jnp.float32),
        scratch_types=[
            pltpu.VMEM((per_w, 128), jnp.int32),
            pltpu.VMEM((per_w, 128, d), jnp.float32),
            pltpu.SemaphoreType.DMA,
        ],
    )
    def k(table_hbm, idx_hbm, out_hbm, idx_v, rows_v, sem):
        wid = lax.axis_index("s") * _NC + lax.axis_index("c")
        base = wid * per_w
        pltpu.sync_copy(idx_hbm.at[pl.ds(base, per_w)], idx_v)

        def sup(j, carry):
            cps = []
            for t in range(u):
                r = j * u + t
                cps.append(pltpu.async_copy(
                    table_hbm.at[idx_v.at[r]], rows_v.at[r], sem))
            for cp in cps:
                cp.wait()
            return carry

        lax.fori_loop(0, nsup, sup, 0)
        pltpu.sync_copy(rows_v, out_hbm.at[pl.ds(base, per_w)])

    return k(table, idx2d)


def _sc_gather_conv(table, idxn, zpad):
    nrow = idxn.shape[0]
    per_w = nrow // _NW
    mesh = plsc.VectorSubcoreMesh(core_axis_name="c", subcore_axis_name="s")

    @functools.partial(
        pl.kernel,
        mesh=mesh,
        compiler_params=pltpu.CompilerParams(use_tc_tiling_on_sc=False),
        out_type=jax.ShapeDtypeStruct((nrow // 13 * 128, 16, 8), jnp.float32),
        scratch_types=[
            pltpu.VMEM((per_w, 128), jnp.int32),
            pltpu.VMEM((per_w, 128, 8), jnp.float32),
            pltpu.SemaphoreType.DMA,
            pltpu.SemaphoreType.DMA,
        ],
    )
    def k(table_hbm, idx_hbm, z_hbm, out_hbm, idx_v, rows_v, sem, wsem):
        wid = lax.axis_index("s") * _NC + lax.axis_index("c")
        base = wid * per_w
        pltpu.sync_copy(idx_hbm.at[pl.ds(base, per_w)], idx_v)

        def grp(g, carry):
            cps = []
            for n in range(_NN):
                r = g * _NN + n
                cps.append(pltpu.async_copy(
                    table_hbm.at[idx_v.at[r]], rows_v.at[r], sem))
            for cp in cps:
                cp.wait()
            wps = []
            site0 = wid * 1024 + g * 128
            for n in range(_NN):
                wps.append(pltpu.async_copy(
                    rows_v.at[g * _NN + n],
                    out_hbm.at[pl.ds(site0, 128), n], wsem))
            for wp in wps:
                wp.wait()
            return carry

        lax.fori_loop(0, per_w // _NN, grp, 0)

    return k(table, idxn, zpad)


def _softplus(x):
    return jnp.maximum(x, 0.0) + jnp.log(1.0 + jnp.exp(-jnp.abs(x)))


def _split_bf16(x):
    hi = x.astype(jnp.bfloat16)
    lo = (x - hi.astype(jnp.float32)).astype(jnp.bfloat16)
    return hi, lo


def _dot3(a, b):
    a_hi, a_lo = _split_bf16(a)
    b_hi, b_lo = _split_bf16(b)
    return (jnp.dot(a_hi, b_hi, preferred_element_type=jnp.float32)
            + (jnp.dot(a_hi, b_lo, preferred_element_type=jnp.float32)
               + jnp.dot(a_lo, b_hi, preferred_element_type=jnp.float32)))


def _group_mean(sp, m_ones):
    s_hi, s_lo = _split_bf16(sp)
    mb = m_ones.astype(jnp.bfloat16)
    return (jnp.dot(s_hi, mb, preferred_element_type=jnp.float32)
            + jnp.dot(s_lo, mb, preferred_element_type=jnp.float32)
            ) * (1.0 / _NG)


def _conv_layer(g, p2, bvec, mm):
    ncol = p2.shape[1]
    o = mm.shape[1]

    def body(g_ref, p_ref, b_ref, m_ref, o_ref):
        gm = g_ref[...][:, :, :_NN * _C].reshape(_B * _BLK, _NN * _C)
        x = _dot3(gm, p_ref[...]) + b_ref[...]
        y = _group_mean(_softplus(x), m_ref[...])
        o_ref[...] = y.reshape(_B, _BLK, o)

    return pl.pallas_call(
        body,
        grid=(_S // _BLK,),
        in_specs=[
            pl.BlockSpec((_B, _BLK, 128), lambda j: (0, j, 0)),
            pl.BlockSpec((_NN * _C, ncol), lambda j: (0, 0)),
            pl.BlockSpec((1, ncol), lambda j: (0, 0)),
            pl.BlockSpec((ncol, o), lambda j: (0, 0)),
        ],
        out_specs=pl.BlockSpec((_B, _BLK, o), lambda j: (0, j, 0)),
        out_shape=jax.ShapeDtypeStruct((_B, _S, o), jnp.float32),
    )(g, p2, bvec, mm)


def _conv_last(g, p2, bvec, mm):
    ncol = p2.shape[1]

    def body(g_ref, p_ref, b_ref, m_ref, o_ref):
        gm = g_ref[...][:, :, :_NN * _C].reshape(_B * _BLK, _NN * _C)
        x = _dot3(gm, p_ref[...]) + b_ref[...]
        y = _group_mean(_softplus(x), m_ref[...])
        y = y.reshape(_B, _BLK, 1)
        cols = [y[b] for b in range(_B)]
        cols.append(jnp.zeros((_BLK, _C - _B), jnp.float32))
        o_ref[...] = jnp.concatenate(cols, axis=1)

    return pl.pallas_call(
        body,
        grid=(_S // _BLK,),
        in_specs=[
            pl.BlockSpec((_B, _BLK, 128), lambda j: (0, j, 0)),
            pl.BlockSpec((_NN * _C, ncol), lambda j: (0, 0)),
            pl.BlockSpec((1, ncol), lambda j: (0, 0)),
            pl.BlockSpec((ncol, 1), lambda j: (0, 0)),
        ],
        out_specs=pl.BlockSpec((_BLK, _C), lambda j: (j, 0)),
        out_shape=jax.ShapeDtypeStruct((_S, _C), jnp.float32),
    )(g, p2, bvec, mm)


def _readout_body(g_ref, sh_ref, k_ref, o_ref):
    blk = g_ref.shape[0]
    oh_t = (lax.broadcasted_iota(jnp.int32, (_NSH, blk), 0) ==
            sh_ref[0]).astype(jnp.float32)
    w2 = lax.dot_general(oh_t, k_ref[...], (((0,), (0,)), ((), ())),
                         preferred_element_type=jnp.float32,
                        precision=lax.Precision.HIGHEST)
    d = lax.dot_general(g_ref[...], w2, (((0,), (0,)), ((), ())),
                        preferred_element_type=jnp.float32,
                        precision=lax.Precision.HIGHEST)

    @pl.when(pl.program_id(0) == 0)
    def _init():
        o_ref[...] = jnp.zeros_like(o_ref)

    o_ref[...] += d


def _readout(g5, shells3, k6):
    return pl.pallas_call(
        _readout_body,
        grid=(_S // _BLK,),
        in_specs=[
            pl.BlockSpec((_BLK, _NN * _C), lambda j: (j, 0)),
            pl.BlockSpec((1, 1, _BLK), lambda j: (j, 0, 0)),
            pl.BlockSpec((_NSH, _NN * _DIM), lambda j: (0, 0)),
        ],
        out_specs=pl.BlockSpec((_NN * _C, _NN * _DIM), lambda j: (0, 0)),
        out_shape=jax.ShapeDtypeStruct((_NN * _C, _NN * _DIM), jnp.float32),
    )(g5, shells3, k6)


def _p2_of(psi, gnnperms):
    psig = psi[:, :, gnnperms]
    p = jnp.transpose(psig, (3, 1, 0, 2))
    cin = p.shape[1]
    if cin < _C:
        p = jnp.pad(p, ((0, 0), (0, _C - cin), (0, 0), (0, 0)))
    o = p.shape[2]
    return p.reshape(_NN * _C, o * _NG)


def kernel(InState, NNsites, GnnPerms, SitesToShells, gdiags,
           Psi0, b0, Psi1, b1, Psi2, b2, Psi3, b3, Psi4, b4, PsiR, VR):
    psis = (Psi0, Psi1, Psi2, Psi3, Psi4)
    biases = (b0, b1, b2, b3, b4)

    nnt = NNsites.T.astype(jnp.int32)
    idx5 = nnt.reshape(_S * _NN // 128, 128)
    idxn = (jnp.arange(_B, dtype=jnp.int32)[:, None, None] * _S
            + nnt[None]).reshape(_B * _S // 128, 128, _NN)
    idxn = jnp.transpose(idxn, (0, 2, 1)).reshape(_B * _S // 128 * _NN, 128)
    zpad = jnp.zeros((512, 3, 8), jnp.float32)

    p2s = [_p2_of(p, GnnPerms) for p in psis]
    bvecs = [jnp.repeat(b, _NG)[None, :] for b in biases]
    mms = [jnp.kron(jnp.eye(p.shape[0], dtype=jnp.float32),
                    jnp.ones((_NG, 1), jnp.float32))
           for p in psis]

    q = jnp.einsum('hd,gde->hge', VR, gdiags)
    psirg = PsiR[:, GnnPerms]
    k6 = (jnp.einsum('hgn,hge->hne', psirg, q) / _NG).reshape(_NSH, _NN * _DIM)

    shells3 = SitesToShells.astype(jnp.int32).reshape(_S // _BLK, 1, _BLK)

    x = jnp.transpose(InState, (0, 2, 1))
    x = jnp.pad(x, ((0, 0), (0, 0), (0, _C - _NSPEC)))

    for li in range(5):
        table = x.reshape(_B * _S, _C)
        g = _sc_gather_conv(table, idxn, zpad)
        g = g.reshape(_B, _S, 128)
        if li < 4:
            x = _conv_layer(g, p2s[li], bvecs[li], mms[li])
        else:
            x = _conv_last(g, p2s[li], bvecs[li], mms[li])

    g5 = _sc_gather(x, idx5, _C).reshape(_S, _NN * _C)
    dmat = _readout(g5, shells3, k6)

    d4 = dmat.reshape(_NN, _C, _NN, _DIM)
    return jnp.einsum('nbne->be', d4)[:_B]
